# baseline (speedup 1.0000x reference)
import dataclasses
import functools

import jax
import jax.numpy as jnp
from jax import lax
from jax.experimental import pallas as pl
from jax.experimental.pallas import tpu as pltpu
from jax.experimental.pallas import tpu_sc as plsc

RADIUS2 = 0.2 * 0.2
NSAMPLE = 64
L = 16
NC = 2
NS = 16


def _ball_query_sc(xt, qt, B, N, M):
    CHUNKS = N // L
    SENTINEL = N + 1
    UNROLL = 16
    QG = 16
    BUFCAP = NSAMPLE + L
    NG = M // QG

    mesh = plsc.VectorSubcoreMesh(core_axis_name="c", subcore_axis_name="s")
    cp = pltpu.CompilerParams()
    if "needs_layout_passes" in pltpu.CompilerParams.__dataclass_fields__:
        cp = dataclasses.replace(cp, needs_layout_passes=False)

    @functools.partial(
        pl.kernel,
        out_type=jax.ShapeDtypeStruct((B * M * NSAMPLE,), jnp.int32),
        mesh=mesh,
        compiler_params=cp,
        scratch_types=[
            pltpu.VMEM((N,), jnp.float32),
            pltpu.VMEM((N,), jnp.float32),
            pltpu.VMEM((N,), jnp.float32),
            pltpu.VMEM((M + L,), jnp.float32),
            pltpu.VMEM((M + L,), jnp.float32),
            pltpu.VMEM((M + L,), jnp.float32),
            pltpu.VMEM((QG * BUFCAP,), jnp.int32),
            pltpu.VMEM((QG * NSAMPLE,), jnp.int32),
            pltpu.SMEM((1,), jnp.int32),
        ],
    )
    def kern(xt_hbm, qt_hbm, out_hbm, xv, yv, zv, qxv, qyv, qzv, buf, oacc, ctr):
        cid = lax.axis_index("c")
        sid = lax.axis_index("s")
        half = sid // 8
        b = cid * 2 + half
        owner = half * 8

        xbase = b * 3 * N
        pltpu.sync_copy(xt_hbm.at[pl.ds(xbase, N)], xv)
        pltpu.sync_copy(xt_hbm.at[pl.ds(xbase + N, N)], yv)
        pltpu.sync_copy(xt_hbm.at[pl.ds(xbase + 2 * N, N)], zv)
        qbase = b * 3 * M
        pltpu.sync_copy(qt_hbm.at[pl.ds(qbase, M)], qxv.at[pl.ds(0, M)])
        pltpu.sync_copy(qt_hbm.at[pl.ds(qbase + M, M)], qyv.at[pl.ds(0, M)])
        pltpu.sync_copy(qt_hbm.at[pl.ds(qbase + 2 * M, M)], qzv.at[pl.ds(0, M)])

        @pl.when(sid % 8 == 0)
        def _():
            ctr[0] = jnp.int32(0)

        plsc.subcore_barrier()

        iota = lax.iota(jnp.int32, L)
        sentv = jnp.full((L,), SENTINEL, jnp.int32)

        capv = [
            jnp.full((L,), i * BUFCAP + NSAMPLE - 1, jnp.int32)
            for i in range(QG)
        ]
        done_total = jnp.int32(sum(i * BUFCAP + NSAMPLE - 1 for i in range(QG)))

        def process_group(g):
            q0 = g * QG
            qxs = [jnp.full((L,), qxv[pl.ds(q0 + i, L)][0]) for i in range(QG)]
            qys = [jnp.full((L,), qyv[pl.ds(q0 + i, L)][0]) for i in range(QG)]
            qzs = [jnp.full((L,), qzv[pl.ds(q0 + i, L)][0]) for i in range(QG)]
            for i in range(QG):
                buf[pl.ds(i * BUFCAP, L)] = sentv

            def cond(carry):
                base = carry[0]
                tot = carry[1]
                for c in carry[2:]:
                    tot = tot + c
                return jnp.logical_and(base < N, tot[0] < done_total)

            def body(carry):
                base = carry[0]
                ccs = list(carry[1:])
                for u in range(UNROLL):
                    xc = xv[pl.ds(base, L)]
                    yc = yv[pl.ds(base, L)]
                    zc = zv[pl.ds(base, L)]
                    idx = base + iota
                    for i in range(QG):
                        dx = qxs[i] - xc
                        dy = qys[i] - yc
                        dz = qzs[i] - zc
                        d2 = dx * dx + dy * dy + dz * dz
                        m = d2 < RADIUS2
                        pos = ccs[i] + plsc.cumsum(m.astype(jnp.int32))
                        plsc.store_scatter(buf, [pos], idx, mask=m)
                        ccs[i] = jnp.minimum(
                            ccs[i] + plsc.all_reduce_population_count(m),
                            capv[i],
                        )
                    base = base + jnp.int32(L)
                return (base, *ccs)

            carry = lax.while_loop(
                cond,
                body,
                (jnp.int32(0),)
                + tuple(
                    jnp.full((L,), i * BUFCAP - 1, jnp.int32)
                    for i in range(QG)
                ),
            )
            for i in range(QG):
                cnt = carry[1 + i][0] - jnp.int32(i * BUFCAP - 1)
                padv = jnp.full((L,), buf[pl.ds(i * BUFCAP, L)][0])
                for k in range(NSAMPLE // L):
                    v = buf[pl.ds(i * BUFCAP + k * L, L)]
                    valid = (k * L + iota) < cnt
                    oacc[pl.ds(i * NSAMPLE + k * L, L)] = jnp.where(
                        valid, v, padv
                    )
            pltpu.sync_copy(
                oacc,
                out_hbm.at[pl.ds((b * M + q0) * NSAMPLE, QG * NSAMPLE)],
            )

        def steal_cond(g):
            return g < NG

        def steal_body(g):
            process_group(g)
            return plsc.fetch_and_add(ctr, 1, subcore_id=owner)

        lax.while_loop(
            steal_cond,
            steal_body,
            plsc.fetch_and_add(ctr, 1, subcore_id=owner),
        )

    return kern(xt, qt)


@jax.jit
def kernel(xyz, new_xyz):
    B, M, _ = new_xyz.shape
    N = xyz.shape[1]
    xt = jnp.transpose(xyz, (0, 2, 1)).reshape(-1)
    qt = jnp.transpose(new_xyz, (0, 2, 1)).reshape(-1)
    out = _ball_query_sc(xt, qt, B, N, M)
    return out.reshape(B, M, NSAMPLE)


# speedup vs baseline: 2.4478x; 2.4478x over previous
import dataclasses
import functools

import jax
import jax.numpy as jnp
from jax import lax
from jax.experimental import pallas as pl
from jax.experimental.pallas import tpu as pltpu
from jax.experimental.pallas import tpu_sc as plsc

RADIUS2 = 0.2 * 0.2
NSAMPLE = 64
L = 16
NC = 2
NS = 16


def _ball_query_sc(xt, qt, B, N, M):
    CHUNKS = N // L
    SENTINEL = N + 1
    UNROLL = 16
    QG = 8
    BUFCAP = NSAMPLE + L
    NG = M // QG

    mesh = plsc.VectorSubcoreMesh(core_axis_name="c", subcore_axis_name="s")
    cp = pltpu.CompilerParams()
    if "needs_layout_passes" in pltpu.CompilerParams.__dataclass_fields__:
        cp = dataclasses.replace(cp, needs_layout_passes=False)

    @functools.partial(
        pl.kernel,
        out_type=jax.ShapeDtypeStruct((B * M * NSAMPLE,), jnp.int32),
        mesh=mesh,
        compiler_params=cp,
        scratch_types=[
            pltpu.VMEM((N,), jnp.float32),
            pltpu.VMEM((N,), jnp.float32),
            pltpu.VMEM((N,), jnp.float32),
            pltpu.VMEM((M + L,), jnp.float32),
            pltpu.VMEM((M + L,), jnp.float32),
            pltpu.VMEM((M + L,), jnp.float32),
            pltpu.VMEM((QG * BUFCAP,), jnp.int32),
            pltpu.VMEM((QG * NSAMPLE,), jnp.int32),
            pltpu.SMEM((1,), jnp.int32),
        ],
    )
    def kern(xt_hbm, qt_hbm, out_hbm, xv, yv, zv, qxv, qyv, qzv, buf, oacc, ctr):
        cid = lax.axis_index("c")
        sid = lax.axis_index("s")
        half = sid // 8
        b = cid * 2 + half
        owner = half * 8

        xbase = b * 3 * N
        pltpu.sync_copy(xt_hbm.at[pl.ds(xbase, N)], xv)
        pltpu.sync_copy(xt_hbm.at[pl.ds(xbase + N, N)], yv)
        pltpu.sync_copy(xt_hbm.at[pl.ds(xbase + 2 * N, N)], zv)
        qbase = b * 3 * M
        pltpu.sync_copy(qt_hbm.at[pl.ds(qbase, M)], qxv.at[pl.ds(0, M)])
        pltpu.sync_copy(qt_hbm.at[pl.ds(qbase + M, M)], qyv.at[pl.ds(0, M)])
        pltpu.sync_copy(qt_hbm.at[pl.ds(qbase + 2 * M, M)], qzv.at[pl.ds(0, M)])

        @pl.when(sid % 8 == 0)
        def _():
            ctr[0] = jnp.int32(0)

        plsc.subcore_barrier()

        iota = lax.iota(jnp.int32, L)
        sentv = jnp.full((L,), SENTINEL, jnp.int32)
        onesv = jnp.full((L,), 1, jnp.int32)

        capv = [
            jnp.full((L,), i * BUFCAP + NSAMPLE - 1, jnp.int32)
            for i in range(QG)
        ]
        done_total = jnp.int32(sum(i * BUFCAP + NSAMPLE - 1 for i in range(QG)))

        def process_group(g):
            q0 = g * QG
            qxs = [jnp.full((L,), qxv[pl.ds(q0 + i, L)][0]) for i in range(QG)]
            qys = [jnp.full((L,), qyv[pl.ds(q0 + i, L)][0]) for i in range(QG)]
            qzs = [jnp.full((L,), qzv[pl.ds(q0 + i, L)][0]) for i in range(QG)]
            for i in range(QG):
                buf[pl.ds(i * BUFCAP, L)] = sentv

            def cond(carry):
                base = carry[0]
                tot = carry[1]
                for c in carry[2:]:
                    tot = tot + c
                return jnp.logical_and(base < N, tot[0] < done_total)

            def body(carry):
                base = carry[0]
                ccs = list(carry[1:])
                for u in range(UNROLL):
                    xc = xv[pl.ds(base, L)]
                    yc = yv[pl.ds(base, L)]
                    zc = zv[pl.ds(base, L)]
                    idx = base + iota
                    for i in range(QG):
                        dx = qxs[i] - xc
                        dy = qys[i] - yc
                        dz = qzs[i] - zc
                        d2 = dx * dx + dy * dy + dz * dz
                        m = d2 < RADIUS2
                        pos = ccs[i] + plsc.cumsum(onesv, mask=m)
                        plsc.store_scatter(buf, [pos], idx, mask=m)
                        ccs[i] = jnp.minimum(
                            ccs[i] + plsc.all_reduce_population_count(m),
                            capv[i],
                        )
                    base = base + jnp.int32(L)
                return (base, *ccs)

            carry = lax.while_loop(
                cond,
                body,
                (jnp.int32(0),)
                + tuple(
                    jnp.full((L,), i * BUFCAP - 1, jnp.int32)
                    for i in range(QG)
                ),
            )
            for i in range(QG):
                cnt = carry[1 + i][0] - jnp.int32(i * BUFCAP - 1)
                padv = jnp.full((L,), buf[pl.ds(i * BUFCAP, L)][0])
                for k in range(NSAMPLE // L):
                    v = buf[pl.ds(i * BUFCAP + k * L, L)]
                    valid = (k * L + iota) < cnt
                    oacc[pl.ds(i * NSAMPLE + k * L, L)] = jnp.where(
                        valid, v, padv
                    )
            pltpu.sync_copy(
                oacc,
                out_hbm.at[pl.ds((b * M + q0) * NSAMPLE, QG * NSAMPLE)],
            )

        def steal_cond(g):
            return g < NG

        def steal_body(g):
            process_group(g)
            return plsc.fetch_and_add(ctr, 1, subcore_id=owner)

        lax.while_loop(
            steal_cond,
            steal_body,
            plsc.fetch_and_add(ctr, 1, subcore_id=owner),
        )

    return kern(xt, qt)


@jax.jit
def kernel(xyz, new_xyz):
    B, M, _ = new_xyz.shape
    N = xyz.shape[1]
    xt = jnp.transpose(xyz, (0, 2, 1)).reshape(-1)
    qt = jnp.transpose(new_xyz, (0, 2, 1)).reshape(-1)
    out = _ball_query_sc(xt, qt, B, N, M)
    return out.reshape(B, M, NSAMPLE)


# speedup vs baseline: 2.4637x; 1.0065x over previous
import dataclasses
import functools

import jax
import jax.numpy as jnp
from jax import lax
from jax.experimental import pallas as pl
from jax.experimental.pallas import tpu as pltpu
from jax.experimental.pallas import tpu_sc as plsc

RADIUS2 = 0.2 * 0.2
NSAMPLE = 64
L = 16
NC = 2
NS = 16


def _ball_query_sc(xt, qt, B, N, M):
    CHUNKS = N // L
    SENTINEL = N + 1
    UNROLL = 16
    QG = 8
    BUFCAP = NSAMPLE + L
    NG = M // QG

    mesh = plsc.VectorSubcoreMesh(core_axis_name="c", subcore_axis_name="s")
    cp = pltpu.CompilerParams()
    if "needs_layout_passes" in pltpu.CompilerParams.__dataclass_fields__:
        cp = dataclasses.replace(cp, needs_layout_passes=False)

    @functools.partial(
        pl.kernel,
        out_type=jax.ShapeDtypeStruct((B * M * NSAMPLE,), jnp.int32),
        mesh=mesh,
        compiler_params=cp,
        scratch_types=[
            pltpu.VMEM((N,), jnp.float32),
            pltpu.VMEM((N,), jnp.float32),
            pltpu.VMEM((N,), jnp.float32),
            pltpu.VMEM((M + L,), jnp.float32),
            pltpu.VMEM((M + L,), jnp.float32),
            pltpu.VMEM((M + L,), jnp.float32),
            pltpu.VMEM((QG * BUFCAP,), jnp.int32),
            pltpu.VMEM((QG * NSAMPLE,), jnp.int32),
            pltpu.SMEM((1,), jnp.int32),
            pltpu.SemaphoreType.DMA,
        ],
    )
    def kern(
        xt_hbm, qt_hbm, out_hbm, xv, yv, zv, qxv, qyv, qzv, buf, oacc, ctr, osem
    ):
        cid = lax.axis_index("c")
        sid = lax.axis_index("s")
        half = sid // 8
        b = cid * 2 + half
        owner = half * 8

        xbase = b * 3 * N
        pltpu.sync_copy(xt_hbm.at[pl.ds(xbase, N)], xv)
        pltpu.sync_copy(xt_hbm.at[pl.ds(xbase + N, N)], yv)
        pltpu.sync_copy(xt_hbm.at[pl.ds(xbase + 2 * N, N)], zv)
        qbase = b * 3 * M
        pltpu.sync_copy(qt_hbm.at[pl.ds(qbase, M)], qxv.at[pl.ds(0, M)])
        pltpu.sync_copy(qt_hbm.at[pl.ds(qbase + M, M)], qyv.at[pl.ds(0, M)])
        pltpu.sync_copy(qt_hbm.at[pl.ds(qbase + 2 * M, M)], qzv.at[pl.ds(0, M)])

        @pl.when(sid % 8 == 0)
        def _():
            ctr[0] = jnp.int32(0)

        plsc.subcore_barrier()

        iota = lax.iota(jnp.int32, L)
        sentv = jnp.full((L,), SENTINEL, jnp.int32)
        onesv = jnp.full((L,), 1, jnp.int32)

        capv = [
            jnp.full((L,), i * BUFCAP + NSAMPLE - 1, jnp.int32)
            for i in range(QG)
        ]
        done_total = jnp.int32(sum(i * BUFCAP + NSAMPLE - 1 for i in range(QG)))

        def process_group(g, started):
            q0 = g * QG
            qxs = [jnp.full((L,), qxv[pl.ds(q0 + i, L)][0]) for i in range(QG)]
            qys = [jnp.full((L,), qyv[pl.ds(q0 + i, L)][0]) for i in range(QG)]
            qzs = [jnp.full((L,), qzv[pl.ds(q0 + i, L)][0]) for i in range(QG)]
            for i in range(QG):
                buf[pl.ds(i * BUFCAP, L)] = sentv

            def cond(carry):
                base = carry[0]
                tot = carry[1]
                for c in carry[2:]:
                    tot = tot + c
                return jnp.logical_and(base < N, tot[0] < done_total)

            def body(carry):
                base = carry[0]
                ccs = list(carry[1:])
                for u in range(UNROLL):
                    xc = xv[pl.ds(base, L)]
                    yc = yv[pl.ds(base, L)]
                    zc = zv[pl.ds(base, L)]
                    idx = base + iota
                    for i in range(QG):
                        dx = qxs[i] - xc
                        dy = qys[i] - yc
                        dz = qzs[i] - zc
                        d2 = dx * dx + dy * dy + dz * dz
                        m = d2 < RADIUS2
                        pos = ccs[i] + plsc.cumsum(onesv, mask=m)
                        plsc.store_scatter(buf, [pos], idx, mask=m)
                        ccs[i] = jnp.minimum(
                            ccs[i] + plsc.all_reduce_population_count(m),
                            capv[i],
                        )
                    base = base + jnp.int32(L)
                return (base, *ccs)

            carry = lax.while_loop(
                cond,
                body,
                (jnp.int32(0),)
                + tuple(
                    jnp.full((L,), i * BUFCAP - 1, jnp.int32)
                    for i in range(QG)
                ),
            )
            @pl.when(started > 0)
            def _():
                pltpu.make_async_copy(
                    oacc, out_hbm.at[pl.ds(0, QG * NSAMPLE)], osem
                ).wait()

            for i in range(QG):
                cnt = carry[1 + i][0] - jnp.int32(i * BUFCAP - 1)
                padv = jnp.full((L,), buf[pl.ds(i * BUFCAP, L)][0])
                for k in range(NSAMPLE // L):
                    v = buf[pl.ds(i * BUFCAP + k * L, L)]
                    valid = (k * L + iota) < cnt
                    oacc[pl.ds(i * NSAMPLE + k * L, L)] = jnp.where(
                        valid, v, padv
                    )
            pltpu.async_copy(
                oacc,
                out_hbm.at[pl.ds((b * M + q0) * NSAMPLE, QG * NSAMPLE)],
                osem,
            )

        def steal_cond(carry):
            return carry[0] < NG

        def steal_body(carry):
            g, started = carry
            process_group(g, started)
            return (
                plsc.fetch_and_add(ctr, 1, subcore_id=owner),
                jnp.int32(1),
            )

        _, started = lax.while_loop(
            steal_cond,
            steal_body,
            (plsc.fetch_and_add(ctr, 1, subcore_id=owner), jnp.int32(0)),
        )

        @pl.when(started > 0)
        def _():
            pltpu.make_async_copy(
                oacc, out_hbm.at[pl.ds(0, QG * NSAMPLE)], osem
            ).wait()

    return kern(xt, qt)


@jax.jit
def kernel(xyz, new_xyz):
    B, M, _ = new_xyz.shape
    N = xyz.shape[1]
    xt = jnp.transpose(xyz, (0, 2, 1)).reshape(-1)
    qt = jnp.transpose(new_xyz, (0, 2, 1)).reshape(-1)
    out = _ball_query_sc(xt, qt, B, N, M)
    return out.reshape(B, M, NSAMPLE)


# speedup vs baseline: 2.6927x; 1.0929x over previous
import dataclasses
import functools

import jax
import jax.numpy as jnp
from jax import lax
from jax.experimental import pallas as pl
from jax.experimental.pallas import tpu as pltpu
from jax.experimental.pallas import tpu_sc as plsc

RADIUS2 = 0.2 * 0.2
NSAMPLE = 64
L = 16
NC = 2
NS = 16


def _ball_query_sc(xt, qt, B, N, M):
    CHUNKS = N // L
    SENTINEL = N + 1
    UNROLL = 16
    QG = 8
    BUFCAP = NSAMPLE + L
    NG = M // QG

    mesh = plsc.VectorSubcoreMesh(core_axis_name="c", subcore_axis_name="s")
    cp = pltpu.CompilerParams()
    if "needs_layout_passes" in pltpu.CompilerParams.__dataclass_fields__:
        cp = dataclasses.replace(cp, needs_layout_passes=False)

    @functools.partial(
        pl.kernel,
        out_type=jax.ShapeDtypeStruct((B * M * NSAMPLE,), jnp.int32),
        mesh=mesh,
        compiler_params=cp,
        scratch_types=[
            pltpu.VMEM((N,), jnp.float32),
            pltpu.VMEM((N,), jnp.float32),
            pltpu.VMEM((N,), jnp.float32),
            pltpu.VMEM((M + L,), jnp.float32),
            pltpu.VMEM((M + L,), jnp.float32),
            pltpu.VMEM((M + L,), jnp.float32),
            pltpu.VMEM((N,), jnp.float32),
            pltpu.VMEM((QG * BUFCAP,), jnp.int32),
            pltpu.VMEM((QG * NSAMPLE,), jnp.int32),
            pltpu.SMEM((1,), jnp.int32),
            pltpu.SemaphoreType.DMA,
        ],
    )
    def kern(
        xt_hbm, qt_hbm, out_hbm, xv, yv, zv, qxv, qyv, qzv, sv, buf, oacc, ctr,
        osem
    ):
        cid = lax.axis_index("c")
        sid = lax.axis_index("s")
        half = sid // 8
        b = cid * 2 + half
        owner = half * 8

        xbase = b * 3 * N
        pltpu.sync_copy(xt_hbm.at[pl.ds(xbase, N)], xv)
        pltpu.sync_copy(xt_hbm.at[pl.ds(xbase + N, N)], yv)
        pltpu.sync_copy(xt_hbm.at[pl.ds(xbase + 2 * N, N)], zv)
        qbase = b * 3 * M
        pltpu.sync_copy(qt_hbm.at[pl.ds(qbase, M)], qxv.at[pl.ds(0, M)])
        pltpu.sync_copy(qt_hbm.at[pl.ds(qbase + M, M)], qyv.at[pl.ds(0, M)])
        pltpu.sync_copy(qt_hbm.at[pl.ds(qbase + 2 * M, M)], qzv.at[pl.ds(0, M)])

        @pl.when(sid % 8 == 0)
        def _():
            ctr[0] = jnp.int32(0)

        plsc.subcore_barrier()

        iota = lax.iota(jnp.int32, L)
        sentv = jnp.full((L,), SENTINEL, jnp.int32)
        onesv = jnp.full((L,), 1, jnp.int32)
        r2v = jnp.full((L,), RADIUS2, jnp.float32)

        @pl.loop(0, N, step=L)
        def _(o):
            xc = xv[pl.ds(o, L)]
            yc = yv[pl.ds(o, L)]
            zc = zv[pl.ds(o, L)]
            sv[pl.ds(o, L)] = xc * xc + yc * yc + zc * zc

        capv = [
            jnp.full((L,), i * BUFCAP + NSAMPLE - 1, jnp.int32)
            for i in range(QG)
        ]
        done_total = jnp.int32(sum(i * BUFCAP + NSAMPLE - 1 for i in range(QG)))

        def process_group(g, started):
            q0 = g * QG
            qxs = [jnp.full((L,), qxv[pl.ds(q0 + i, L)][0]) for i in range(QG)]
            qys = [jnp.full((L,), qyv[pl.ds(q0 + i, L)][0]) for i in range(QG)]
            qzs = [jnp.full((L,), qzv[pl.ds(q0 + i, L)][0]) for i in range(QG)]
            q2xs = [q + q for q in qxs]
            q2ys = [q + q for q in qys]
            q2zs = [q + q for q in qzs]
            rhs = [
                r2v - qxs[i] * qxs[i] - qys[i] * qys[i] - qzs[i] * qzs[i]
                for i in range(QG)
            ]
            for i in range(QG):
                buf[pl.ds(i * BUFCAP, L)] = sentv

            def cond(carry):
                base = carry[0]
                tot = carry[1]
                for c in carry[2:]:
                    tot = tot + c
                return jnp.logical_and(base < N, tot[0] < done_total)

            def body(carry):
                base = carry[0]
                ccs = list(carry[1:])
                for u in range(UNROLL):
                    xc = xv[pl.ds(base, L)]
                    yc = yv[pl.ds(base, L)]
                    zc = zv[pl.ds(base, L)]
                    sc = sv[pl.ds(base, L)]
                    idx = base + iota
                    for i in range(QG):
                        t = sc - q2xs[i] * xc
                        t = t - q2ys[i] * yc
                        t = t - q2zs[i] * zc
                        m = t < rhs[i]
                        pos = ccs[i] + plsc.cumsum(onesv, mask=m)
                        plsc.store_scatter(buf, [pos], idx, mask=m)
                        ccs[i] = jnp.minimum(
                            ccs[i] + plsc.all_reduce_population_count(m),
                            capv[i],
                        )
                    base = base + jnp.int32(L)
                return (base, *ccs)

            carry = lax.while_loop(
                cond,
                body,
                (jnp.int32(0),)
                + tuple(
                    jnp.full((L,), i * BUFCAP - 1, jnp.int32)
                    for i in range(QG)
                ),
            )
            @pl.when(started > 0)
            def _():
                pltpu.make_async_copy(
                    oacc, out_hbm.at[pl.ds(0, QG * NSAMPLE)], osem
                ).wait()

            for i in range(QG):
                cnt = carry[1 + i][0] - jnp.int32(i * BUFCAP - 1)
                padv = jnp.full((L,), buf[pl.ds(i * BUFCAP, L)][0])
                for k in range(NSAMPLE // L):
                    v = buf[pl.ds(i * BUFCAP + k * L, L)]
                    valid = (k * L + iota) < cnt
                    oacc[pl.ds(i * NSAMPLE + k * L, L)] = jnp.where(
                        valid, v, padv
                    )
            pltpu.async_copy(
                oacc,
                out_hbm.at[pl.ds((b * M + q0) * NSAMPLE, QG * NSAMPLE)],
                osem,
            )

        def steal_cond(carry):
            return carry[0] < NG

        def steal_body(carry):
            g, started = carry
            process_group(g, started)
            return (
                plsc.fetch_and_add(ctr, 1, subcore_id=owner),
                jnp.int32(1),
            )

        _, started = lax.while_loop(
            steal_cond,
            steal_body,
            (plsc.fetch_and_add(ctr, 1, subcore_id=owner), jnp.int32(0)),
        )

        @pl.when(started > 0)
        def _():
            pltpu.make_async_copy(
                oacc, out_hbm.at[pl.ds(0, QG * NSAMPLE)], osem
            ).wait()

    return kern(xt, qt)


@jax.jit
def kernel(xyz, new_xyz):
    B, M, _ = new_xyz.shape
    N = xyz.shape[1]
    xt = jnp.transpose(xyz, (0, 2, 1)).reshape(-1)
    qt = jnp.transpose(new_xyz, (0, 2, 1)).reshape(-1)
    out = _ball_query_sc(xt, qt, B, N, M)
    return out.reshape(B, M, NSAMPLE)
